# baseline (speedup 1.0000x reference)
import jax
import jax.numpy as jnp
from jax.experimental import pallas as pl
from jax.experimental.pallas import tpu as pltpu

E = 64
D = 1024
DFF = 4096
BLK = 128
DFFT = 2048
K = DFF // DFFT


def _ffn_seg_kernel(seg_e, seg_k, seg_b, seg_r0, seg_r1,
                    x_ref, cent_ref, lns_ref, lnb_ref,
                    w1_ref, b1_ref, w2_ref, b2_ref,
                    out_ref, acc_scr):
    t = pl.program_id(0)
    k = seg_k[t]
    row0 = seg_b[t] * BLK

    x = x_ref[...]
    mu = jnp.mean(x, axis=1, keepdims=True)
    var = jnp.mean((x - mu) * (x - mu), axis=1, keepdims=True)
    xn = (x - mu) * jax.lax.rsqrt(var + 1e-5)
    xn = xn * lns_ref[0, 0][None, :] + lnb_ref[0, 0][None, :]
    xn = xn.astype(jnp.bfloat16)

    w1 = w1_ref[0].astype(jnp.bfloat16)
    b1 = b1_ref[0, 0]
    h = jax.lax.dot_general(xn, w1, (((1,), (1,)), ((), ())),
                            preferred_element_type=jnp.float32)
    h = jnp.maximum(h + b1[None, :], 0.0).astype(jnp.bfloat16)
    w2 = w2_ref[0].astype(jnp.bfloat16)
    part = jax.lax.dot_general(h, w2, (((1,), (0,)), ((), ())),
                               preferred_element_type=jnp.float32)

    @pl.when(k == 0)
    def _():
        acc_scr[pl.ds(row0, BLK), :] = part

    @pl.when(k != 0)
    def _():
        acc_scr[pl.ds(row0, BLK), :] += part

    @pl.when(k == K - 1)
    def _():
        c = cent_ref[0, 0]
        logit = jnp.sum(x * c[None, :], axis=1, keepdims=True)
        alpha = jax.nn.sigmoid(logit)
        y = x + alpha * (acc_scr[pl.ds(row0, BLK), :] + b2_ref[0, 0][None, :])
        rows = jax.lax.broadcasted_iota(jnp.int32, (BLK, 1), 0)
        mask = (rows >= seg_r0[t]) & (rows < seg_r1[t])
        out_ref[pl.ds(row0, BLK), :] = jnp.where(
            mask, y, out_ref[pl.ds(row0, BLK), :])


def _expert_ffn(routed, seg_e, seg_k, seg_b, seg_r0, seg_r1,
                expert_centroids, ln_scale, ln_bias, W1, b1, W2, b2):
    T = routed.shape[0]
    nstep = seg_e.shape[0]
    cent3 = expert_centroids.reshape(E, 1, D)
    lns3 = ln_scale.reshape(E, 1, D)
    lnb3 = ln_bias.reshape(E, 1, D)
    b1_3 = b1.reshape(E, 1, DFF)
    b2_3 = b2.reshape(E, 1, D)

    grid_spec = pltpu.PrefetchScalarGridSpec(
        num_scalar_prefetch=5,
        grid=(nstep,),
        in_specs=[
            pl.BlockSpec((BLK, D), lambda t, se, sk, sb, r0, r1: (sb[t], 0)),
            pl.BlockSpec((1, 1, D), lambda t, se, sk, sb, r0, r1: (se[t], 0, 0)),
            pl.BlockSpec((1, 1, D), lambda t, se, sk, sb, r0, r1: (se[t], 0, 0)),
            pl.BlockSpec((1, 1, D), lambda t, se, sk, sb, r0, r1: (se[t], 0, 0)),
            pl.BlockSpec((1, DFFT, D), lambda t, se, sk, sb, r0, r1: (se[t], sk[t], 0)),
            pl.BlockSpec((1, 1, DFFT), lambda t, se, sk, sb, r0, r1: (se[t], 0, sk[t])),
            pl.BlockSpec((1, DFFT, D), lambda t, se, sk, sb, r0, r1: (se[t], sk[t], 0)),
            pl.BlockSpec((1, 1, D), lambda t, se, sk, sb, r0, r1: (se[t], 0, 0)),
        ],
        out_specs=pl.BlockSpec((T, D), lambda t, se, sk, sb, r0, r1: (0, 0)),
        scratch_shapes=[pltpu.VMEM((T, D), jnp.float32)],
    )
    return pl.pallas_call(
        _ffn_seg_kernel,
        grid_spec=grid_spec,
        out_shape=jax.ShapeDtypeStruct((T, D), jnp.float32),
        compiler_params=pltpu.CompilerParams(
            dimension_semantics=("arbitrary",),
        ),
    )(seg_e, seg_k, seg_b, seg_r0, seg_r1,
      routed, cent3, lns3, lnb3, W1, b1_3, W2, b2_3)


def kernel(input_features, expert_centroids, ln_scale, ln_bias, W1, b1, W2, b2):
    shape = input_features.shape
    x = input_features.reshape(-1, shape[-1])
    T = x.shape[0]
    nseg = (T // BLK) + E - 1
    nstep = K * nseg

    scores = x @ expert_centroids.T
    tok_e = jnp.argmax(scores, axis=1).astype(jnp.int32)
    order = jnp.argsort(tok_e).astype(jnp.int32)
    routed = x[order]

    counts = jnp.bincount(tok_e, length=E)
    off = jnp.concatenate([jnp.zeros((1,), jnp.int32),
                           jnp.cumsum(counts).astype(jnp.int32)])
    cnt = off[1:] - off[:-1]
    fb = off[:-1] // BLK
    lb = jnp.where(cnt > 0, (off[1:] - 1) // BLK, fb - 1)
    m = jnp.where(cnt > 0, lb - fb + 1, 0)
    cumf = jnp.cumsum(K * m).astype(jnp.int32)
    flat_start = jnp.concatenate([jnp.zeros((1,), jnp.int32), cumf])
    total = flat_start[E]

    t_idx = jnp.arange(nstep, dtype=jnp.int32)
    e_t = jnp.searchsorted(cumf, t_idx, side='right').astype(jnp.int32)
    e_t = jnp.minimum(e_t, E - 1)
    local = t_idx - flat_start[e_t]
    m_t = jnp.maximum(m[e_t], 1)
    k_t = local // m_t
    b_t = fb[e_t] + (local % m_t)
    r0 = jnp.maximum(off[e_t], b_t * BLK) - b_t * BLK
    r1 = jnp.minimum(off[e_t + 1], (b_t + 1) * BLK) - b_t * BLK
    valid = t_idx < total
    last = jnp.maximum(total - 1, 0)
    e_t = jnp.where(valid, e_t, e_t[last]).astype(jnp.int32)
    k_t = jnp.where(valid, k_t, K - 1).astype(jnp.int32)
    b_t = jnp.where(valid, b_t, b_t[last]).astype(jnp.int32)
    r0 = jnp.where(valid, r0, 0).astype(jnp.int32)
    r1 = jnp.where(valid, r1, 0).astype(jnp.int32)

    W2t = W2.transpose(0, 2, 1)
    out_sorted = _expert_ffn(routed, e_t, k_t, b_t, r0, r1,
                             expert_centroids, ln_scale, ln_bias, W1, b1, W2t, b2)

    inv = jnp.zeros((T,), jnp.int32).at[order].set(
        jnp.arange(T, dtype=jnp.int32))
    result = out_sorted[inv]
    return result.reshape(shape)


# speedup vs baseline: 1.0017x; 1.0017x over previous
import jax
import jax.numpy as jnp
from jax.experimental import pallas as pl
from jax.experimental.pallas import tpu as pltpu

E = 64
D = 1024
DFF = 4096
BLK = 128
DFFT = 2048
K1 = DFF // DFFT
DSL = 256
K2 = D // DSL


def _ffn_seg_kernel(seg_p, seg_kd, seg_b, seg_r0, seg_r1, seg_e, seg_w1k, seg_w2d,
                    x_ref, xsl_ref, cent_ref, lns_ref, lnb_ref,
                    w1_ref, b1_ref, w2_ref, b2_ref,
                    out_ref, h_scr):
    t = pl.program_id(0)
    p = seg_p[t]
    kd = seg_kd[t]
    row0 = seg_b[t] * BLK

    x = x_ref[...]

    @pl.when(p == 0)
    def _():
        mu = jnp.mean(x, axis=1, keepdims=True)
        var = jnp.mean((x - mu) * (x - mu), axis=1, keepdims=True)
        xn = (x - mu) * jax.lax.rsqrt(var + 1e-5)
        xn = xn * lns_ref[0, 0][None, :] + lnb_ref[0, 0][None, :]
        xn = xn.astype(jnp.bfloat16)
        w1 = w1_ref[0].astype(jnp.bfloat16)
        b1 = b1_ref[0, 0]
        h = jax.lax.dot_general(xn, w1, (((1,), (1,)), ((), ())),
                                preferred_element_type=jnp.float32)
        h = jnp.maximum(h + b1[None, :], 0.0)
        h_scr[kd, pl.ds(row0, BLK), :] = h.astype(jnp.bfloat16)

    @pl.when(p == 1)
    def _():
        w2 = w2_ref[0].astype(jnp.bfloat16)
        part = jnp.zeros((BLK, DSL), jnp.float32)
        for kk in range(K1):
            hk = h_scr[kk, pl.ds(row0, BLK), :]
            part += jax.lax.dot_general(
                hk, w2[:, kk * DFFT:(kk + 1) * DFFT], (((1,), (1,)), ((), ())),
                preferred_element_type=jnp.float32)
        c = cent_ref[0, 0]
        logit = jnp.sum(x * c[None, :], axis=1, keepdims=True)
        alpha = jax.nn.sigmoid(logit)
        xd = xsl_ref[:, kd, :]
        b2d = b2_ref[0, kd, :][None, :]
        y = xd + alpha * (part + b2d)
        rows = jax.lax.broadcasted_iota(jnp.int32, (BLK, 1), 0)
        mask = (rows >= seg_r0[t]) & (rows < seg_r1[t])
        cur = out_ref[pl.ds(row0, BLK), pl.ds(kd, 1), :]
        out_ref[pl.ds(row0, BLK), pl.ds(kd, 1), :] = jnp.where(
            mask[:, :, None], y[:, None, :], cur)


def _expert_ffn(routed, scal, expert_centroids, ln_scale, ln_bias, W1, b1, W2, b2):
    T = routed.shape[0]
    nstep = scal[0].shape[0]
    cent3 = expert_centroids.reshape(E, 1, D)
    lns3 = ln_scale.reshape(E, 1, D)
    lnb3 = ln_bias.reshape(E, 1, D)
    b1_3 = b1.reshape(E, 1, DFF)
    b2_3 = b2.reshape(E, K2, DSL)
    routed3 = routed.reshape(T, K2, DSL)

    grid_spec = pltpu.PrefetchScalarGridSpec(
        num_scalar_prefetch=8,
        grid=(nstep,),
        in_specs=[
            pl.BlockSpec((BLK, D),
                         lambda t, p, kd, sb, r0, r1, se, w1k, w2d: (sb[t], 0)),
            pl.BlockSpec((BLK, K2, DSL),
                         lambda t, p, kd, sb, r0, r1, se, w1k, w2d: (sb[t], 0, 0)),
            pl.BlockSpec((1, 1, D),
                         lambda t, p, kd, sb, r0, r1, se, w1k, w2d: (se[t], 0, 0)),
            pl.BlockSpec((1, 1, D),
                         lambda t, p, kd, sb, r0, r1, se, w1k, w2d: (se[t], 0, 0)),
            pl.BlockSpec((1, 1, D),
                         lambda t, p, kd, sb, r0, r1, se, w1k, w2d: (se[t], 0, 0)),
            pl.BlockSpec((1, DFFT, D),
                         lambda t, p, kd, sb, r0, r1, se, w1k, w2d: (se[t], w1k[t], 0)),
            pl.BlockSpec((1, 1, DFFT),
                         lambda t, p, kd, sb, r0, r1, se, w1k, w2d: (se[t], 0, w1k[t])),
            pl.BlockSpec((1, DSL, DFF),
                         lambda t, p, kd, sb, r0, r1, se, w1k, w2d: (se[t], w2d[t], 0)),
            pl.BlockSpec((1, K2, DSL),
                         lambda t, p, kd, sb, r0, r1, se, w1k, w2d: (se[t], 0, 0)),
        ],
        out_specs=pl.BlockSpec((T, K2, DSL),
                               lambda t, p, kd, sb, r0, r1, se, w1k, w2d: (0, 0, 0)),
        scratch_shapes=[pltpu.VMEM((K1, T, DFFT), jnp.bfloat16)],
    )
    out = pl.pallas_call(
        _ffn_seg_kernel,
        grid_spec=grid_spec,
        out_shape=jax.ShapeDtypeStruct((T, K2, DSL), jnp.float32),
        compiler_params=pltpu.CompilerParams(
            dimension_semantics=("arbitrary",),
        ),
    )(*scal, routed, routed3, cent3, lns3, lnb3, W1, b1_3, W2, b2_3)
    return out.reshape(T, D)


def kernel(input_features, expert_centroids, ln_scale, ln_bias, W1, b1, W2, b2):
    shape = input_features.shape
    x = input_features.reshape(-1, shape[-1])
    T = x.shape[0]
    nseg = (T // BLK) + E - 1
    P = K1 + K2
    nstep = P * nseg

    scores = x @ expert_centroids.T
    tok_e = jnp.argmax(scores, axis=1).astype(jnp.int32)
    order = jnp.argsort(tok_e).astype(jnp.int32)
    routed = x[order]

    counts = jnp.bincount(tok_e, length=E)
    off = jnp.concatenate([jnp.zeros((1,), jnp.int32),
                           jnp.cumsum(counts).astype(jnp.int32)])
    cnt = off[1:] - off[:-1]
    fb = off[:-1] // BLK
    lb = jnp.where(cnt > 0, (off[1:] - 1) // BLK, fb - 1)
    m = jnp.where(cnt > 0, lb - fb + 1, 0)
    cumf = jnp.cumsum(P * m).astype(jnp.int32)
    flat_start = jnp.concatenate([jnp.zeros((1,), jnp.int32), cumf])
    total = flat_start[E]

    t_idx = jnp.arange(nstep, dtype=jnp.int32)
    e_t = jnp.searchsorted(cumf, t_idx, side='right').astype(jnp.int32)
    e_t = jnp.minimum(e_t, E - 1)
    local = t_idx - flat_start[e_t]
    m_t = jnp.maximum(m[e_t], 1)
    phase_idx = local // m_t
    p_t = (phase_idx >= K1).astype(jnp.int32)
    kd_t = jnp.where(p_t == 0, phase_idx, phase_idx - K1)
    b_t = fb[e_t] + (local % m_t)
    w1k_t = jnp.where(p_t == 0, kd_t, K1 - 1)
    w2d_t = jnp.where(p_t == 0, 0, kd_t)
    r0 = jnp.maximum(off[e_t], b_t * BLK) - b_t * BLK
    r1 = jnp.minimum(off[e_t + 1], (b_t + 1) * BLK) - b_t * BLK
    valid = t_idx < total
    last = jnp.maximum(total - 1, 0)
    e_t = jnp.where(valid, e_t, e_t[last]).astype(jnp.int32)
    b_t = jnp.where(valid, b_t, b_t[last]).astype(jnp.int32)
    p_t = jnp.where(valid, p_t, 1).astype(jnp.int32)
    kd_t = jnp.where(valid, kd_t, K2 - 1).astype(jnp.int32)
    w1k_t = jnp.where(valid, w1k_t, K1 - 1).astype(jnp.int32)
    w2d_t = jnp.where(valid, w2d_t, K2 - 1).astype(jnp.int32)
    r0 = jnp.where(valid, r0, 0).astype(jnp.int32)
    r1 = jnp.where(valid, r1, 0).astype(jnp.int32)

    scal = (p_t, kd_t, b_t, r0, r1, e_t, w1k_t, w2d_t)
    out_sorted = _expert_ffn(routed, scal,
                             expert_centroids, ln_scale, ln_bias, W1, b1, W2, b2)

    inv = jnp.zeros((T,), jnp.int32).at[order].set(
        jnp.arange(T, dtype=jnp.int32))
    result = out_sorted[inv]
    return result.reshape(shape)


# speedup vs baseline: 1.7420x; 1.7391x over previous
import jax
import jax.numpy as jnp
from jax.experimental import pallas as pl
from jax.experimental.pallas import tpu as pltpu

E = 64
D = 1024
DFF = 4096
BLK = 128
DFFT = 2048
K = DFF // DFFT


def _ffn_seg_kernel(seg_e, seg_k, seg_b, seg_r0, seg_r1,
                    x_ref, cent_ref, lns_ref, lnb_ref,
                    w1_ref, b1_ref, w2_ref, b2_ref,
                    out_ref, acc_scr):
    t = pl.program_id(0)
    k = seg_k[t]
    row0 = seg_b[t] * BLK

    x = x_ref[...]
    mu = jnp.mean(x, axis=1, keepdims=True)
    var = jnp.mean((x - mu) * (x - mu), axis=1, keepdims=True)
    xn = (x - mu) * jax.lax.rsqrt(var + 1e-5)
    xn = xn * lns_ref[0, 0][None, :] + lnb_ref[0, 0][None, :]
    xn = xn.astype(jnp.bfloat16)

    w1 = w1_ref[0].astype(jnp.bfloat16)
    b1 = b1_ref[0, 0]
    h = jax.lax.dot_general(xn, w1, (((1,), (1,)), ((), ())),
                            preferred_element_type=jnp.float32)
    h = jnp.maximum(h + b1[None, :], 0.0).astype(jnp.bfloat16)
    w2 = w2_ref[0].astype(jnp.bfloat16)
    part = jax.lax.dot_general(h, w2, (((1,), (1,)), ((), ())),
                               preferred_element_type=jnp.float32)

    @pl.when(k == 0)
    def _():
        acc_scr[pl.ds(row0, BLK), :] = part

    @pl.when(k != 0)
    def _():
        acc_scr[pl.ds(row0, BLK), :] += part

    @pl.when(k == K - 1)
    def _():
        c = cent_ref[0, 0]
        logit = jnp.sum(x * c[None, :], axis=1, keepdims=True)
        alpha = jax.nn.sigmoid(logit)
        y = x + alpha * (acc_scr[pl.ds(row0, BLK), :] + b2_ref[0, 0][None, :])
        rows = jax.lax.broadcasted_iota(jnp.int32, (BLK, 1), 0)
        mask = (rows >= seg_r0[t]) & (rows < seg_r1[t])
        out_ref[pl.ds(row0, BLK), :] = jnp.where(
            mask, y, out_ref[pl.ds(row0, BLK), :])


def _expert_ffn(routed, seg_e, seg_k, seg_b, seg_r0, seg_r1,
                expert_centroids, ln_scale, ln_bias, W1, b1, W2, b2):
    T = routed.shape[0]
    nstep = seg_e.shape[0]
    cent3 = expert_centroids.reshape(E, 1, D)
    lns3 = ln_scale.reshape(E, 1, D)
    lnb3 = ln_bias.reshape(E, 1, D)
    b1_3 = b1.reshape(E, 1, DFF)
    b2_3 = b2.reshape(E, 1, D)

    grid_spec = pltpu.PrefetchScalarGridSpec(
        num_scalar_prefetch=5,
        grid=(nstep,),
        in_specs=[
            pl.BlockSpec((BLK, D), lambda t, se, sk, sb, r0, r1: (sb[t], 0)),
            pl.BlockSpec((1, 1, D), lambda t, se, sk, sb, r0, r1: (se[t], 0, 0)),
            pl.BlockSpec((1, 1, D), lambda t, se, sk, sb, r0, r1: (se[t], 0, 0)),
            pl.BlockSpec((1, 1, D), lambda t, se, sk, sb, r0, r1: (se[t], 0, 0)),
            pl.BlockSpec((1, DFFT, D), lambda t, se, sk, sb, r0, r1: (se[t], sk[t], 0)),
            pl.BlockSpec((1, 1, DFFT), lambda t, se, sk, sb, r0, r1: (se[t], 0, sk[t])),
            pl.BlockSpec((1, D, DFFT), lambda t, se, sk, sb, r0, r1: (se[t], 0, sk[t])),
            pl.BlockSpec((1, 1, D), lambda t, se, sk, sb, r0, r1: (se[t], 0, 0)),
        ],
        out_specs=pl.BlockSpec((T, D), lambda t, se, sk, sb, r0, r1: (0, 0)),
        scratch_shapes=[pltpu.VMEM((T, D), jnp.float32)],
    )
    return pl.pallas_call(
        _ffn_seg_kernel,
        grid_spec=grid_spec,
        out_shape=jax.ShapeDtypeStruct((T, D), jnp.float32),
        compiler_params=pltpu.CompilerParams(
            dimension_semantics=("arbitrary",),
        ),
    )(seg_e, seg_k, seg_b, seg_r0, seg_r1,
      routed, cent3, lns3, lnb3, W1, b1_3, W2, b2_3)


def kernel(input_features, expert_centroids, ln_scale, ln_bias, W1, b1, W2, b2):
    shape = input_features.shape
    x = input_features.reshape(-1, shape[-1])
    T = x.shape[0]
    nseg = (T // BLK) + E - 1
    nstep = K * nseg

    scores = x @ expert_centroids.T
    tok_e = jnp.argmax(scores, axis=1).astype(jnp.int32)
    onehot = (tok_e[:, None] == jnp.arange(E, dtype=jnp.int32)[None, :])
    ranks = jnp.cumsum(onehot.astype(jnp.int32), axis=0)
    counts = ranks[-1]
    off = jnp.concatenate([jnp.zeros((1,), jnp.int32),
                           jnp.cumsum(counts).astype(jnp.int32)])
    dest = off[tok_e] + jnp.take_along_axis(ranks, tok_e[:, None], axis=1)[:, 0] - 1
    order = jnp.zeros((T,), jnp.int32).at[dest].set(jnp.arange(T, dtype=jnp.int32))
    routed = x[order]
    cnt = off[1:] - off[:-1]
    fb = off[:-1] // BLK
    lb = jnp.where(cnt > 0, (off[1:] - 1) // BLK, fb - 1)
    m = jnp.where(cnt > 0, lb - fb + 1, 0)
    cumf = jnp.cumsum(K * m).astype(jnp.int32)
    flat_start = jnp.concatenate([jnp.zeros((1,), jnp.int32), cumf])
    total = flat_start[E]

    t_idx = jnp.arange(nstep, dtype=jnp.int32)
    e_t = jnp.searchsorted(cumf, t_idx, side='right').astype(jnp.int32)
    e_t = jnp.minimum(e_t, E - 1)
    local = t_idx - flat_start[e_t]
    m_t = jnp.maximum(m[e_t], 1)
    k_t = local // m_t
    b_t = fb[e_t] + (local % m_t)
    r0 = jnp.maximum(off[e_t], b_t * BLK) - b_t * BLK
    r1 = jnp.minimum(off[e_t + 1], (b_t + 1) * BLK) - b_t * BLK
    valid = t_idx < total
    last = jnp.maximum(total - 1, 0)
    e_t = jnp.where(valid, e_t, e_t[last]).astype(jnp.int32)
    k_t = jnp.where(valid, k_t, K - 1).astype(jnp.int32)
    b_t = jnp.where(valid, b_t, b_t[last]).astype(jnp.int32)
    r0 = jnp.where(valid, r0, 0).astype(jnp.int32)
    r1 = jnp.where(valid, r1, 0).astype(jnp.int32)

    out_sorted = _expert_ffn(routed, e_t, k_t, b_t, r0, r1,
                             expert_centroids, ln_scale, ln_bias, W1, b1, W2, b2)

    result = out_sorted[dest]
    return result.reshape(shape)


# speedup vs baseline: 1.8046x; 1.0359x over previous
import jax
import jax.numpy as jnp
from jax.experimental import pallas as pl
from jax.experimental.pallas import tpu as pltpu

E = 64
D = 1024
DFF = 4096
BLK = 128
DFFT = 2048
K = DFF // DFFT


def _ffn_seg_kernel(seg_e, seg_k, seg_b, seg_r0, seg_r1,
                    x_ref, cent_ref, lns_ref, lnb_ref,
                    w1_ref, b1_ref, w2a_ref, w2b_ref, b2_ref,
                    out_ref, acc_scr):
    t = pl.program_id(0)
    k = seg_k[t]
    row0 = seg_b[t] * BLK

    x = x_ref[...]
    mu = jnp.mean(x, axis=1, keepdims=True)
    var = jnp.mean((x - mu) * (x - mu), axis=1, keepdims=True)
    xn = (x - mu) * jax.lax.rsqrt(var + 1e-5)
    xn = xn * lns_ref[0, 0][None, :] + lnb_ref[0, 0][None, :]
    xn = xn.astype(jnp.bfloat16)

    w1 = w1_ref[0].astype(jnp.bfloat16)
    b1 = b1_ref[0, 0]
    h = jax.lax.dot_general(xn, w1, (((1,), (1,)), ((), ())),
                            preferred_element_type=jnp.float32)
    h = jnp.maximum(h + b1[None, :], 0.0).astype(jnp.bfloat16)
    w2a = w2a_ref[0, 0].astype(jnp.bfloat16)
    w2b = w2b_ref[0, 0].astype(jnp.bfloat16)
    part_a = jax.lax.dot_general(h, w2a, (((1,), (1,)), ((), ())),
                                 preferred_element_type=jnp.float32)
    part_b = jax.lax.dot_general(h, w2b, (((1,), (1,)), ((), ())),
                                 preferred_element_type=jnp.float32)

    @pl.when(k == 0)
    def _():
        acc_scr[pl.ds(row0, BLK), 0:D // 2] = part_a
        acc_scr[pl.ds(row0, BLK), D // 2:D] = part_b

    @pl.when(k != 0)
    def _():
        acc_scr[pl.ds(row0, BLK), 0:D // 2] += part_a
        acc_scr[pl.ds(row0, BLK), D // 2:D] += part_b

    @pl.when(k == K - 1)
    def _():
        c = cent_ref[0, 0]
        logit = jnp.sum(x * c[None, :], axis=1, keepdims=True)
        alpha = jax.nn.sigmoid(logit)
        y = x + alpha * (acc_scr[pl.ds(row0, BLK), :] + b2_ref[0, 0][None, :])
        rows = jax.lax.broadcasted_iota(jnp.int32, (BLK, 1), 0)
        mask = (rows >= seg_r0[t]) & (rows < seg_r1[t])
        out_ref[pl.ds(row0, BLK), :] = jnp.where(
            mask, y, out_ref[pl.ds(row0, BLK), :])


def _expert_ffn(routed, seg_e, seg_k, seg_b, seg_r0, seg_r1,
                expert_centroids, ln_scale, ln_bias, W1, b1, W2, b2):
    T = routed.shape[0]
    nstep = seg_e.shape[0]
    cent3 = expert_centroids.reshape(E, 1, D)
    lns3 = ln_scale.reshape(E, 1, D)
    lnb3 = ln_bias.reshape(E, 1, D)
    b1_3 = b1.reshape(E, 1, DFF)
    b2_3 = b2.reshape(E, 1, D)

    grid_spec = pltpu.PrefetchScalarGridSpec(
        num_scalar_prefetch=5,
        grid=(nstep,),
        in_specs=[
            pl.BlockSpec((BLK, D), lambda t, se, sk, sb, r0, r1: (sb[t], 0)),
            pl.BlockSpec((1, 1, D), lambda t, se, sk, sb, r0, r1: (se[t], 0, 0)),
            pl.BlockSpec((1, 1, D), lambda t, se, sk, sb, r0, r1: (se[t], 0, 0)),
            pl.BlockSpec((1, 1, D), lambda t, se, sk, sb, r0, r1: (se[t], 0, 0)),
            pl.BlockSpec((1, DFFT, D), lambda t, se, sk, sb, r0, r1: (se[t], sk[t], 0)),
            pl.BlockSpec((1, 1, DFFT), lambda t, se, sk, sb, r0, r1: (se[t], 0, sk[t])),
            pl.BlockSpec((1, 1, D // 2, DFFT),
                         lambda t, se, sk, sb, r0, r1: (se[t], 0, 0, sk[t])),
            pl.BlockSpec((1, 1, D // 2, DFFT),
                         lambda t, se, sk, sb, r0, r1: (se[t], 1, 0, sk[t])),
            pl.BlockSpec((1, 1, D), lambda t, se, sk, sb, r0, r1: (se[t], 0, 0)),
        ],
        out_specs=pl.BlockSpec((T, D), lambda t, se, sk, sb, r0, r1: (0, 0)),
        scratch_shapes=[pltpu.VMEM((T, D), jnp.float32)],
    )
    W2r = W2.reshape(E, 2, D // 2, DFF)
    return pl.pallas_call(
        _ffn_seg_kernel,
        grid_spec=grid_spec,
        out_shape=jax.ShapeDtypeStruct((T, D), jnp.float32),
        compiler_params=pltpu.CompilerParams(
            dimension_semantics=("arbitrary",),
        ),
    )(seg_e, seg_k, seg_b, seg_r0, seg_r1,
      routed, cent3, lns3, lnb3, W1, b1_3, W2r, W2r, b2_3)


def kernel(input_features, expert_centroids, ln_scale, ln_bias, W1, b1, W2, b2):
    shape = input_features.shape
    x = input_features.reshape(-1, shape[-1])
    T = x.shape[0]
    nseg = (T // BLK) + E - 1
    nstep = K * nseg

    scores = x @ expert_centroids.T
    tok_e = jnp.argmax(scores, axis=1).astype(jnp.int32)
    order = jnp.argsort(tok_e).astype(jnp.int32)
    routed = x[order]

    counts = jnp.bincount(tok_e, length=E)
    off = jnp.concatenate([jnp.zeros((1,), jnp.int32),
                           jnp.cumsum(counts).astype(jnp.int32)])
    cnt = off[1:] - off[:-1]
    fb = off[:-1] // BLK
    lb = jnp.where(cnt > 0, (off[1:] - 1) // BLK, fb - 1)
    m = jnp.where(cnt > 0, lb - fb + 1, 0)
    cumf = jnp.cumsum(K * m).astype(jnp.int32)
    flat_start = jnp.concatenate([jnp.zeros((1,), jnp.int32), cumf])
    total = flat_start[E]

    t_idx = jnp.arange(nstep, dtype=jnp.int32)
    e_t = jnp.searchsorted(cumf, t_idx, side='right').astype(jnp.int32)
    e_t = jnp.minimum(e_t, E - 1)
    local = t_idx - flat_start[e_t]
    m_t = jnp.maximum(m[e_t], 1)
    k_t = local // m_t
    b_t = fb[e_t] + (local % m_t)
    r0 = jnp.maximum(off[e_t], b_t * BLK) - b_t * BLK
    r1 = jnp.minimum(off[e_t + 1], (b_t + 1) * BLK) - b_t * BLK
    valid = t_idx < total
    last = jnp.maximum(total - 1, 0)
    e_t = jnp.where(valid, e_t, e_t[last]).astype(jnp.int32)
    k_t = jnp.where(valid, k_t, K - 1).astype(jnp.int32)
    b_t = jnp.where(valid, b_t, b_t[last]).astype(jnp.int32)
    r0 = jnp.where(valid, r0, 0).astype(jnp.int32)
    r1 = jnp.where(valid, r1, 0).astype(jnp.int32)

    out_sorted = _expert_ffn(routed, e_t, k_t, b_t, r0, r1,
                             expert_centroids, ln_scale, ln_bias, W1, b1, W2, b2)

    inv = jnp.zeros((T,), jnp.int32).at[order].set(
        jnp.arange(T, dtype=jnp.int32))
    result = out_sorted[inv]
    return result.reshape(shape)
